# Initial kernel scaffold; baseline (speedup 1.0000x reference)
import jax
import jax.numpy as jnp
from jax.experimental import pallas as pl


def kernel(inputs, E0, E1, W2, b2, W3, b3, W4, b4, W5, b5, W6, b6, W7, b7):
    raise NotImplementedError("write your pallas kernel here")


# speedup vs baseline: 1.3521x; 1.3521x over previous
import functools

import jax
import jax.numpy as jnp
from jax import lax
from jax.experimental import pallas as pl
from jax.experimental.pallas import tpu as pltpu
from jax.experimental.pallas import tpu_sc as plsc

B = 1024
W = 200
HIST = 150
FUT = W - HIST
D = 64
NW = 32
CH = 128
E1_ROWS = B * W
NCH = E1_ROWS // (NW * CH)
SROWS = B // NW

def _sc_gather_body(e0, i0, e1, idx2d, static_o, e1_o, si_v, sr_v, ix_v, rows_v, sem):
    c = lax.axis_index("c")
    s = lax.axis_index("s")
    wid = s * 2 + c
    sb = wid * SROWS
    pltpu.sync_copy(i0.at[pl.ds(sb, SROWS)], si_v)
    pltpu.async_copy(e0.at[si_v], sr_v, sem).wait()
    pltpu.sync_copy(sr_v, static_o.at[pl.ds(sb, SROWS)])
    pltpu.sync_copy(idx2d.at[wid], ix_v)
    base = wid * (NCH * CH)

    def chunk(ci, carry):
        pltpu.async_copy(e1.at[ix_v.at[ci]], rows_v, sem).wait()
        pltpu.sync_copy(rows_v, e1_o.at[pl.ds(base + ci * CH, CH)])
        return carry

    lax.fori_loop(0, NCH, chunk, 0)


@functools.cache
def _get_sc_gather():
    mesh = plsc.VectorSubcoreMesh(core_axis_name="c", subcore_axis_name="s")
    return pl.kernel(
        _sc_gather_body,
        out_type=(
            jax.ShapeDtypeStruct((B, D), jnp.float32),
            jax.ShapeDtypeStruct((E1_ROWS, D), jnp.float32),
        ),
        mesh=mesh,
        scratch_types=[
            pltpu.VMEM((SROWS,), jnp.int32),
            pltpu.VMEM((SROWS, D), jnp.float32),
            pltpu.VMEM((NCH, CH), jnp.int32),
            pltpu.VMEM((CH, D), jnp.float32),
            pltpu.SemaphoreType.DMA,
        ],
        compiler_params=pltpu.CompilerParams(use_tc_tiling_on_sc=False),
    )


def _tc_hist_body(x_ref, e1_ref, wp_ref, bp_ref, out_ref):
    xb = x_ref[...]
    out_ref[:, D:2 * D] = e1_ref[...]
    for slot, ch in ((0, 7), (2, 5), (3, 6), (4, 2), (5, 3), (6, 4)):
        out_ref[:, slot * D:(slot + 1) * D] = (
            xb[:, ch:ch + 1] * wp_ref[ch:ch + 1, :] + bp_ref[ch:ch + 1, :]
        )


def _tc_fut_body(x_ref, e1_ref, wp_ref, bp_ref, out_ref):
    xb = x_ref[...]
    out_ref[:, 0:D] = e1_ref[...]
    for slot, ch in ((1, 5), (2, 6)):
        out_ref[:, slot * D:(slot + 1) * D] = (
            xb[:, ch:ch + 1] * wp_ref[ch:ch + 1, :] + bp_ref[ch:ch + 1, :]
        )


_RBS = 1024


def kernel(inputs, E0, E1, W2, b2, W3, b3, W4, b4, W5, b5, W6, b6, W7, b7):
    ii = inputs.astype(jnp.int32)
    i0 = ii[:, 0, 0]
    idx_all = jnp.concatenate(
        [ii[:, :HIST, 1].reshape(-1), ii[:, HIST:, 1].reshape(-1)]
    ).reshape(NW, NCH, CH)

    static2d, e1_all = _get_sc_gather()(E0, i0, E1, idx_all)

    x_h = inputs[:, :HIST, :].reshape(B * HIST, 8)
    x_f = inputs[:, HIST:, :].reshape(B * FUT, 8)
    wp = jnp.concatenate(
        [jnp.zeros((2, D), jnp.float32), W2, W3, W4, W5, W6, W7], axis=0
    )
    bp = jnp.stack([b2, b2, b2, b3, b4, b5, b6, b7], axis=0)
    bp = bp.at[0:2].set(0.0)

    n_h = B * HIST // _RBS
    n_f = B * FUT // _RBS

    hist = pl.pallas_call(
        _tc_hist_body,
        grid=(n_h,),
        in_specs=[
            pl.BlockSpec((_RBS, 8), lambda r: (r, 0)),
            pl.BlockSpec((_RBS, D), lambda r: (r, 0)),
            pl.BlockSpec((8, D), lambda r: (0, 0)),
            pl.BlockSpec((8, D), lambda r: (0, 0)),
        ],
        out_specs=pl.BlockSpec((_RBS, 7 * D), lambda r: (r, 0)),
        out_shape=jax.ShapeDtypeStruct((B * HIST, 7 * D), jnp.float32),
    )(x_h, e1_all, wp, bp)

    fut = pl.pallas_call(
        _tc_fut_body,
        grid=(n_f,),
        in_specs=[
            pl.BlockSpec((_RBS, 8), lambda r: (r, 0)),
            pl.BlockSpec((_RBS, D), lambda r: (r + n_h, 0)),
            pl.BlockSpec((8, D), lambda r: (0, 0)),
            pl.BlockSpec((8, D), lambda r: (0, 0)),
        ],
        out_specs=pl.BlockSpec((_RBS, 3 * D), lambda r: (r, 0)),
        out_shape=jax.ShapeDtypeStruct((B * FUT, 3 * D), jnp.float32),
    )(x_f, e1_all, wp, bp)

    return (
        static2d.reshape(B, 1, D),
        hist.reshape(B, HIST, 7, D),
        fut.reshape(B, FUT, 3, D),
    )


# speedup vs baseline: 4.1748x; 3.0876x over previous
import functools

import jax
import jax.numpy as jnp
from jax import lax
from jax.experimental import pallas as pl
from jax.experimental.pallas import tpu as pltpu
from jax.experimental.pallas import tpu_sc as plsc

B = 1024
W = 200
HIST = 150
FUT = W - HIST
D = 64
NW = 32
CH = 128
E1_ROWS = B * W
NCH = E1_ROWS // (NW * CH)
SROWS = B // NW

def _sc_gather_body(e0, i0, e1, idx2d, static_o, e1_o, si_v, sr_v, ix_v, rows_v, sem):
    c = lax.axis_index("c")
    s = lax.axis_index("s")
    wid = s * 2 + c
    sb = wid * SROWS
    pltpu.sync_copy(i0.at[pl.ds(sb, SROWS)], si_v)
    pltpu.async_copy(e0.at[si_v], sr_v, sem).wait()
    pltpu.sync_copy(sr_v, static_o.at[pl.ds(sb, SROWS)])
    pltpu.sync_copy(idx2d.at[wid], ix_v)
    base = wid * (NCH * CH)

    def chunk(ci, carry):
        pltpu.async_copy(e1.at[ix_v.at[ci]], rows_v, sem).wait()
        pltpu.sync_copy(rows_v, e1_o.at[pl.ds(base + ci * CH, CH)])
        return carry

    lax.fori_loop(0, NCH, chunk, 0)


@functools.cache
def _get_sc_gather():
    mesh = plsc.VectorSubcoreMesh(core_axis_name="c", subcore_axis_name="s")
    return pl.kernel(
        _sc_gather_body,
        out_type=(
            jax.ShapeDtypeStruct((B, D), jnp.float32),
            jax.ShapeDtypeStruct((E1_ROWS, D), jnp.float32),
        ),
        mesh=mesh,
        scratch_types=[
            pltpu.VMEM((SROWS,), jnp.int32),
            pltpu.VMEM((SROWS, D), jnp.float32),
            pltpu.VMEM((NCH, CH), jnp.int32),
            pltpu.VMEM((CH, D), jnp.float32),
            pltpu.SemaphoreType.DMA,
        ],
        compiler_params=pltpu.CompilerParams(use_tc_tiling_on_sc=False),
    )


def _tc_hist_body(x_ref, e1_ref, wt_ref, bt_ref, out_ref):
    xb = x_ref[0]
    e1b = e1_ref[...]
    out_ref[0, 1] = jnp.swapaxes(e1b, 0, 1)
    for slot, ch in ((0, 7), (2, 5), (3, 6), (4, 2), (5, 3), (6, 4)):
        out_ref[0, slot] = (
            wt_ref[:, ch:ch + 1] * xb[ch:ch + 1, :] + bt_ref[:, ch:ch + 1]
        )


def _tc_fut_body(x_ref, e1_ref, wt_ref, bt_ref, out_ref):
    xb = x_ref[0]
    e1b = e1_ref[...]
    out_ref[0, 0] = jnp.swapaxes(e1b, 0, 1)
    for slot, ch in ((1, 5), (2, 6)):
        out_ref[0, slot] = (
            wt_ref[:, ch:ch + 1] * xb[ch:ch + 1, :] + bt_ref[:, ch:ch + 1]
        )


def kernel(inputs, E0, E1, W2, b2, W3, b3, W4, b4, W5, b5, W6, b6, W7, b7):
    ii = inputs.astype(jnp.int32)
    i0 = ii[:, 0, 0]
    idx_all = jnp.concatenate(
        [ii[:, :HIST, 1].T.reshape(-1), ii[:, HIST:, 1].T.reshape(-1)]
    ).reshape(NW, NCH, CH)

    static2d, e1_all = _get_sc_gather()(E0, i0, E1, idx_all)

    xT = jnp.transpose(inputs, (1, 2, 0))
    wp = jnp.concatenate(
        [jnp.zeros((2, D), jnp.float32), W2, W3, W4, W5, W6, W7], axis=0
    )
    bp = jnp.stack([b2, b2, b2, b3, b4, b5, b6, b7], axis=0)
    bp = bp.at[0:2].set(0.0)
    wt = wp.T
    bt = bp.T

    hist_t = pl.pallas_call(
        _tc_hist_body,
        grid=(HIST,),
        in_specs=[
            pl.BlockSpec((1, 8, B), lambda t: (t, 0, 0)),
            pl.BlockSpec((B, D), lambda t: (t, 0)),
            pl.BlockSpec((D, 8), lambda t: (0, 0)),
            pl.BlockSpec((D, 8), lambda t: (0, 0)),
        ],
        out_specs=pl.BlockSpec((1, 7, D, B), lambda t: (t, 0, 0, 0)),
        out_shape=jax.ShapeDtypeStruct((HIST, 7, D, B), jnp.float32),
    )(xT, e1_all, wt, bt)

    fut_t = pl.pallas_call(
        _tc_fut_body,
        grid=(FUT,),
        in_specs=[
            pl.BlockSpec((1, 8, B), lambda t: (t + HIST, 0, 0)),
            pl.BlockSpec((B, D), lambda t: (t + HIST, 0)),
            pl.BlockSpec((D, 8), lambda t: (0, 0)),
            pl.BlockSpec((D, 8), lambda t: (0, 0)),
        ],
        out_specs=pl.BlockSpec((1, 3, D, B), lambda t: (t, 0, 0, 0)),
        out_shape=jax.ShapeDtypeStruct((FUT, 3, D, B), jnp.float32),
    )(xT, e1_all, wt, bt)

    return (
        static2d.reshape(B, 1, D),
        jnp.transpose(hist_t, (3, 0, 1, 2)),
        jnp.transpose(fut_t, (3, 0, 1, 2)),
    )
